# baseline (speedup 1.0000x reference)
import functools
import jax
import jax.numpy as jnp
from jax import lax
from jax.experimental import pallas as pl
from jax.experimental.pallas import tpu as pltpu
from jax.experimental.pallas import tpu_sc as plsc

_B = 4
_S = 4096
_D = 1024
_NW = 32
_SPW = _S // _NW
_R = 16
_CHUNK = _R * _D


def _sc_body(x_hbm, pos_hbm, out_hbm, xbuf, pbuf):
    wid = lax.axis_index("s") * 2 + lax.axis_index("c")
    for j in range(_SPW // _R):
        seq_base = wid * _SPW + j * _R
        pltpu.sync_copy(pos_hbm.at[pl.ds(seq_base * _D, _CHUNK)], pbuf)
        for b in range(_B):
            off = (b * _S + seq_base) * _D
            pltpu.sync_copy(x_hbm.at[pl.ds(off, _CHUNK)], xbuf)

            def add_one(i, _):
                sl = pl.ds(i * 16, 16)
                xbuf[sl] = xbuf[sl] + pbuf[sl]
                return 0

            lax.fori_loop(0, _CHUNK // 16, add_one, 0)
            pltpu.sync_copy(xbuf, out_hbm.at[pl.ds(off, _CHUNK)])


def kernel(x, pos_table):
    batch, seq_len, d_model = x.shape
    xf = x.reshape(batch * seq_len * d_model)
    posf = pos_table.reshape(-1)[: seq_len * d_model]

    mesh = plsc.VectorSubcoreMesh(core_axis_name="c", subcore_axis_name="s")
    k = functools.partial(
        pl.kernel,
        mesh=mesh,
        out_type=jax.ShapeDtypeStruct((batch * seq_len * d_model,), x.dtype),
        scratch_types=[
            pltpu.VMEM((_CHUNK,), jnp.float32),
            pltpu.VMEM((_CHUNK,), jnp.float32),
        ],
    )(_sc_body)
    out = k(xf, posf)
    return out.reshape(batch, seq_len, d_model)


# speedup vs baseline: 1.6970x; 1.6970x over previous
import functools
import jax
import jax.numpy as jnp
from jax import lax
from jax.experimental import pallas as pl
from jax.experimental.pallas import tpu as pltpu
from jax.experimental.pallas import tpu_sc as plsc

_B = 4
_S = 4096
_D = 1024
_NW = 32
_SPW = _S // _NW
_R = 16
_CHUNK = _R * _D
_NJ = _SPW // _R
_NG = _NJ * _B


def _sc_body(x_hbm, pos_hbm, out_hbm,
             xb0, xb1, pb0, pb1, sin0, sin1, sout0, sout1, sp0, sp1):
    xbufs = (xb0, xb1)
    pbufs = (pb0, pb1)
    sins = (sin0, sin1)
    souts = (sout0, sout1)
    sps = (sp0, sp1)

    wid = lax.axis_index("s") * 2 + lax.axis_index("c")
    base_seq = wid * _SPW

    def x_off(g):
        j, b = divmod(g, _B)
        return (b * _S + base_seq + j * _R) * _D

    in_d = [None] * _NG
    out_d = [None] * _NG
    p_d = [None] * _NJ

    p_d[0] = pltpu.async_copy(
        pos_hbm.at[pl.ds(base_seq * _D, _CHUNK)], pbufs[0], sps[0])
    in_d[0] = pltpu.async_copy(
        x_hbm.at[pl.ds(x_off(0), _CHUNK)], xbufs[0], sins[0])

    for g in range(_NG):
        j, b = divmod(g, _B)
        buf = g % 2
        in_d[g].wait()
        if b == 0:
            p_d[j].wait()
            if j + 1 < _NJ:
                p_d[j + 1] = pltpu.async_copy(
                    pos_hbm.at[pl.ds((base_seq + (j + 1) * _R) * _D, _CHUNK)],
                    pbufs[(j + 1) % 2], sps[(j + 1) % 2])
        if g + 1 < _NG:
            if g - 1 >= 0:
                out_d[g - 1].wait()
            in_d[g + 1] = pltpu.async_copy(
                x_hbm.at[pl.ds(x_off(g + 1), _CHUNK)],
                xbufs[(g + 1) % 2], sins[(g + 1) % 2])

        xb = xbufs[buf]
        pb = pbufs[j % 2]

        @plsc.parallel_loop(0, _CHUNK, step=16, unroll=8)
        def _add(i):
            sl = pl.ds(i, 16)
            xb[sl] = xb[sl] + pb[sl]

        out_d[g] = pltpu.async_copy(
            xb, out_hbm.at[pl.ds(x_off(g), _CHUNK)], souts[buf])

    out_d[_NG - 2].wait()
    out_d[_NG - 1].wait()


def kernel(x, pos_table):
    batch, seq_len, d_model = x.shape
    xf = x.reshape(batch * seq_len * d_model)
    posf = pos_table.reshape(-1)[: seq_len * d_model]

    mesh = plsc.VectorSubcoreMesh(core_axis_name="c", subcore_axis_name="s")
    k = functools.partial(
        pl.kernel,
        mesh=mesh,
        out_type=jax.ShapeDtypeStruct((batch * seq_len * d_model,), x.dtype),
        scratch_types=[
            pltpu.VMEM((_CHUNK,), jnp.float32),
            pltpu.VMEM((_CHUNK,), jnp.float32),
            pltpu.VMEM((_CHUNK,), jnp.float32),
            pltpu.VMEM((_CHUNK,), jnp.float32),
            pltpu.SemaphoreType.DMA,
            pltpu.SemaphoreType.DMA,
            pltpu.SemaphoreType.DMA,
            pltpu.SemaphoreType.DMA,
            pltpu.SemaphoreType.DMA,
            pltpu.SemaphoreType.DMA,
        ],
    )(_sc_body)
    out = k(xf, posf)
    return out.reshape(batch, seq_len, d_model)
